# baseline (speedup 1.0000x reference)
import functools

import jax
import jax.numpy as jnp
from jax import lax
from jax.experimental import pallas as pl
from jax.experimental.pallas import tpu as pltpu
from jax.experimental.pallas import tpu_sc as plsc

E = 320000
D = 128
G = 512
NC = 2
NS = 16
NW = NC * NS
L = 16
CH = 80
ROWS_PW = E // NW
CPT = ROWS_PW // CH
SEG_PT = G // NS

_mesh = plsc.VectorSubcoreMesh(core_axis_name="c", subcore_axis_name="s")


@functools.partial(
    pl.kernel,
    out_type=(
        jax.ShapeDtypeStruct((NC, G, D), jnp.float32),
        jax.ShapeDtypeStruct((NC, G, D), jnp.float32),
    ),
    mesh=_mesh,
    scratch_types=dict(
        ids_v=pltpu.VMEM((CPT, CH), jnp.int32),
        rows_v=pltpu.VMEM((2, CH, D), jnp.float32),
        sems=pltpu.SemaphoreType.DMA((2,)),
        ones_v=pltpu.VMEM((CH, D), jnp.float32),
        zseg_v=pltpu.VMEM((SEG_PT, D), jnp.float32),
        zcnt_v=pltpu.VMEM((SEG_PT, D), jnp.float32),
        facc_v=pltpu.VMEM((SEG_PT, D), jnp.float32),
        fcnt_v=pltpu.VMEM((SEG_PT, D), jnp.float32),
        acc_sh=pltpu.VMEM_SHARED((G, D), jnp.float32),
        cnt_sh=pltpu.VMEM_SHARED((G, D), jnp.float32),
    ),
)
def _seg_sum(feat_hbm, ids_hbm, sums_hbm, cnt_hbm, *, ids_v, rows_v, sems,
             ones_v, zseg_v, zcnt_v, facc_v, fcnt_v, acc_sh, cnt_sh):
    c = lax.axis_index("c")
    s = lax.axis_index("s")
    w = c * NS + s

    ones16 = jnp.ones((L,), jnp.float32)
    zeros16 = jnp.zeros((L,), jnp.float32)
    for i in range(CH):
        for j in range(D // L):
            ones_v[i, pl.ds(j * L, L)] = ones16
    for i in range(SEG_PT):
        for j in range(D // L):
            zseg_v[i, pl.ds(j * L, L)] = zeros16
            zcnt_v[i, pl.ds(j * L, L)] = zeros16

    pltpu.sync_copy(zseg_v, acc_sh.at[pl.ds(s * SEG_PT, SEG_PT)])
    pltpu.sync_copy(zcnt_v, cnt_sh.at[pl.ds(s * SEG_PT, SEG_PT)])
    plsc.subcore_barrier()

    pltpu.sync_copy(ids_hbm.at[w], ids_v)

    row0 = w * ROWS_PW

    def _feat_chunk(i):
        return feat_hbm.at[pl.ds(row0 + i * CH, CH)]

    def _step(i, cur, cur_sem, nxt, nxt_sem):
        pltpu.make_async_copy(_feat_chunk(i), cur, cur_sem).wait()

        @pl.when(i + 1 < CPT)
        def _():
            pltpu.async_copy(_feat_chunk(i + 1), nxt, nxt_sem)

        pltpu.sync_copy(cur, acc_sh.at[ids_v.at[i]], add=True)
        pltpu.sync_copy(ones_v, cnt_sh.at[ids_v.at[i]], add=True)

    pltpu.async_copy(_feat_chunk(0), rows_v.at[0], sems.at[0])

    def chunk(i, carry):
        @pl.when(i % 2 == 0)
        def _():
            _step(i, rows_v.at[0], sems.at[0], rows_v.at[1], sems.at[1])

        @pl.when(i % 2 == 1)
        def _():
            _step(i, rows_v.at[1], sems.at[1], rows_v.at[0], sems.at[0])

        return carry

    lax.fori_loop(0, CPT, chunk, 0)
    plsc.subcore_barrier()

    g0 = s * SEG_PT
    pltpu.sync_copy(acc_sh.at[pl.ds(g0, SEG_PT)], facc_v)
    pltpu.sync_copy(cnt_sh.at[pl.ds(g0, SEG_PT)], fcnt_v)
    pltpu.sync_copy(facc_v, sums_hbm.at[c, pl.ds(g0, SEG_PT)])
    pltpu.sync_copy(fcnt_v, cnt_hbm.at[c, pl.ds(g0, SEG_PT)])


def _finalize_body(sums_ref, cnt_ref, out_ref):
    total = sums_ref[0] + sums_ref[1]
    cnt = cnt_ref[0] + cnt_ref[1]
    denom = jnp.maximum(cnt[:, 0:1], 1.0)
    out_ref[...] = total / denom


_finalize = pl.pallas_call(
    _finalize_body,
    out_shape=jax.ShapeDtypeStruct((G, D), jnp.float32),
)


def kernel(feat, segment_ids, num_graphs):
    del num_graphs
    ids = segment_ids.astype(jnp.int32).reshape(NW, CPT, CH)
    sums, cnt = _seg_sum(feat, ids)
    return _finalize(sums, cnt)


# speedup vs baseline: 1.4591x; 1.4591x over previous
import functools

import jax
import jax.numpy as jnp
from jax import lax
from jax.experimental import pallas as pl
from jax.experimental.pallas import tpu as pltpu
from jax.experimental.pallas import tpu_sc as plsc

E = 320000
D = 128
G = 512
NC = 2
NS = 16
NW = NC * NS
L = 16
CH = 80
ROWS_PW = E // NW
CPT = ROWS_PW // CH
SEG_PT = G // NS
IDR = E // D
HB = 8

_mesh = plsc.VectorSubcoreMesh(core_axis_name="c", subcore_axis_name="s")


@functools.partial(
    pl.kernel,
    out_type=jax.ShapeDtypeStruct((NC, G, D), jnp.float32),
    mesh=_mesh,
    scratch_types=dict(
        ids_v=pltpu.VMEM((CPT, CH), jnp.int32),
        rows_v=pltpu.VMEM((2, CH, D), jnp.float32),
        sems=pltpu.SemaphoreType.DMA((2,)),
        zseg_v=pltpu.VMEM((SEG_PT, D), jnp.float32),
        facc_v=pltpu.VMEM((SEG_PT, D), jnp.float32),
        acc_sh=pltpu.VMEM_SHARED((G, D), jnp.float32),
    ),
)
def _seg_sum(feat_hbm, ids_hbm, sums_hbm, *, ids_v, rows_v, sems, zseg_v,
             facc_v, acc_sh):
    c = lax.axis_index("c")
    s = lax.axis_index("s")
    w = c * NS + s

    zeros16 = jnp.zeros((L,), jnp.float32)
    for i in range(SEG_PT):
        for j in range(D // L):
            zseg_v[i, pl.ds(j * L, L)] = zeros16

    pltpu.sync_copy(zseg_v, acc_sh.at[pl.ds(s * SEG_PT, SEG_PT)])
    plsc.subcore_barrier()

    pltpu.sync_copy(ids_hbm.at[w], ids_v)

    row0 = w * ROWS_PW

    def _feat_chunk(i):
        return feat_hbm.at[pl.ds(row0 + i * CH, CH)]

    def _step(i, cur, cur_sem, nxt, nxt_sem):
        pltpu.make_async_copy(_feat_chunk(i), cur, cur_sem).wait()

        @pl.when(i + 1 < CPT)
        def _():
            pltpu.async_copy(_feat_chunk(i + 1), nxt, nxt_sem)

        pltpu.sync_copy(cur, acc_sh.at[ids_v.at[i]], add=True)

    pltpu.async_copy(_feat_chunk(0), rows_v.at[0], sems.at[0])

    def chunk(i, carry):
        @pl.when(i % 2 == 0)
        def _():
            _step(i, rows_v.at[0], sems.at[0], rows_v.at[1], sems.at[1])

        @pl.when(i % 2 == 1)
        def _():
            _step(i, rows_v.at[1], sems.at[1], rows_v.at[0], sems.at[0])

        return carry

    lax.fori_loop(0, CPT, chunk, 0)
    plsc.subcore_barrier()

    g0 = s * SEG_PT
    pltpu.sync_copy(acc_sh.at[pl.ds(g0, SEG_PT)], facc_v)
    pltpu.sync_copy(facc_v, sums_hbm.at[c, pl.ds(g0, SEG_PT)])


def _hist_body(ids_ref, acc_ref):
    gcol = lax.broadcasted_iota(jnp.int32, (G, 1), 0)

    def blk(r, acc):
        rows = ids_ref[pl.ds(r * HB, HB), :]
        for q in range(HB):
            acc = acc + (gcol == rows[q:q + 1, :]).astype(jnp.float32)
        return acc

    acc_ref[...] = lax.fori_loop(0, IDR // HB, blk,
                                 jnp.zeros((G, D), jnp.float32))


_hist = pl.pallas_call(
    _hist_body,
    out_shape=jax.ShapeDtypeStruct((G, D), jnp.float32),
)


def _finalize_body(sums_ref, acc_ref, out_ref):
    total = sums_ref[0] + sums_ref[1]
    cnt = jnp.sum(acc_ref[...], axis=1, keepdims=True)
    out_ref[...] = total / jnp.maximum(cnt, 1.0)


_finalize = pl.pallas_call(
    _finalize_body,
    out_shape=jax.ShapeDtypeStruct((G, D), jnp.float32),
)


def kernel(feat, segment_ids, num_graphs):
    del num_graphs
    ids = segment_ids.astype(jnp.int32)
    sums = _seg_sum(feat, ids.reshape(NW, CPT, CH))
    acc = _hist(ids.reshape(IDR, D))
    return _finalize(sums, acc)


# speedup vs baseline: 1.4592x; 1.0001x over previous
import functools

import jax
import jax.numpy as jnp
from jax import lax
from jax.experimental import pallas as pl
from jax.experimental.pallas import tpu as pltpu
from jax.experimental.pallas import tpu_sc as plsc

E = 320000
D = 128
G = 512
NC = 2
NS = 16
NW = NC * NS
L = 16
CH = 80
ROWS_PW = E // NW
CPT = ROWS_PW // CH
SEG_PT = G // NS
IDR = E // D
HB = 4

_mesh = plsc.VectorSubcoreMesh(core_axis_name="c", subcore_axis_name="s")


@functools.partial(
    pl.kernel,
    out_type=jax.ShapeDtypeStruct((NC, G, D), jnp.float32),
    mesh=_mesh,
    scratch_types=dict(
        ids_v=pltpu.VMEM((CPT, CH), jnp.int32),
        rows_v=pltpu.VMEM((2, CH, D), jnp.float32),
        sems=pltpu.SemaphoreType.DMA((2,)),
        zseg_v=pltpu.VMEM((SEG_PT, D), jnp.float32),
        facc_v=pltpu.VMEM((SEG_PT, D), jnp.float32),
        acc_sh=pltpu.VMEM_SHARED((G, D), jnp.float32),
    ),
)
def _seg_sum(feat_hbm, ids_hbm, sums_hbm, *, ids_v, rows_v, sems, zseg_v,
             facc_v, acc_sh):
    c = lax.axis_index("c")
    s = lax.axis_index("s")
    w = c * NS + s

    zeros16 = jnp.zeros((L,), jnp.float32)
    for i in range(SEG_PT):
        for j in range(D // L):
            zseg_v[i, pl.ds(j * L, L)] = zeros16

    pltpu.sync_copy(zseg_v, acc_sh.at[pl.ds(s * SEG_PT, SEG_PT)])
    plsc.subcore_barrier()

    pltpu.sync_copy(ids_hbm.at[w], ids_v)

    row0 = w * ROWS_PW

    def _feat_chunk(i):
        return feat_hbm.at[pl.ds(row0 + i * CH, CH)]

    def _step(i, cur, cur_sem, nxt, nxt_sem):
        pltpu.make_async_copy(_feat_chunk(i), cur, cur_sem).wait()

        @pl.when(i + 1 < CPT)
        def _():
            pltpu.async_copy(_feat_chunk(i + 1), nxt, nxt_sem)

        pltpu.sync_copy(cur, acc_sh.at[ids_v.at[i]], add=True)

    pltpu.async_copy(_feat_chunk(0), rows_v.at[0], sems.at[0])

    def chunk(i, carry):
        @pl.when(i % 2 == 0)
        def _():
            _step(i, rows_v.at[0], sems.at[0], rows_v.at[1], sems.at[1])

        @pl.when(i % 2 == 1)
        def _():
            _step(i, rows_v.at[1], sems.at[1], rows_v.at[0], sems.at[0])

        return carry

    lax.fori_loop(0, CPT, chunk, 0)
    plsc.subcore_barrier()

    g0 = s * SEG_PT
    pltpu.sync_copy(acc_sh.at[pl.ds(g0, SEG_PT)], facc_v)
    pltpu.sync_copy(facc_v, sums_hbm.at[c, pl.ds(g0, SEG_PT)])


def _hist_body(ids_ref, acc_ref):
    gcol = lax.broadcasted_iota(jnp.int32, (G, 1), 0)

    def blk(r, acc):
        rows = ids_ref[pl.ds(r * HB, HB), :]
        for q in range(HB):
            acc = acc + (gcol == rows[q:q + 1, :]).astype(jnp.float32)
        return acc

    acc_ref[...] = lax.fori_loop(0, IDR // HB, blk,
                                 jnp.zeros((G, D), jnp.float32))


_hist = pl.pallas_call(
    _hist_body,
    out_shape=jax.ShapeDtypeStruct((G, D), jnp.float32),
)


def _finalize_body(sums_ref, acc_ref, out_ref):
    total = sums_ref[0] + sums_ref[1]
    cnt = jnp.sum(acc_ref[...], axis=1, keepdims=True)
    out_ref[...] = total / jnp.maximum(cnt, 1.0)


_finalize = pl.pallas_call(
    _finalize_body,
    out_shape=jax.ShapeDtypeStruct((G, D), jnp.float32),
)


def kernel(feat, segment_ids, num_graphs):
    del num_graphs
    ids = segment_ids.astype(jnp.int32)
    sums = _seg_sum(feat, ids.reshape(NW, CPT, CH))
    acc = _hist(ids.reshape(IDR, D))
    return _finalize(sums, acc)


# speedup vs baseline: 1.5261x; 1.0458x over previous
import functools

import jax
import jax.numpy as jnp
from jax import lax
from jax.experimental import pallas as pl
from jax.experimental.pallas import tpu as pltpu
from jax.experimental.pallas import tpu_sc as plsc

E = 320000
D = 128
G = 512
NC = 2
NS = 16
NW = NC * NS
L = 16
CH = 80
ROWS_PW = E // NW
CPT = ROWS_PW // CH
SEG_PT = G // NS
IDR = E // D
HB = 4

_mesh = plsc.VectorSubcoreMesh(core_axis_name="c", subcore_axis_name="s")


@functools.partial(
    pl.kernel,
    out_type=jax.ShapeDtypeStruct((NC, G, D), jnp.float32),
    mesh=_mesh,
    scratch_types=dict(
        ids_v=pltpu.VMEM((CPT, CH), jnp.int32),
        rows_v=pltpu.VMEM((2, CH, D), jnp.float32),
        sems=pltpu.SemaphoreType.DMA((2,)),
        zseg_v=pltpu.VMEM((SEG_PT, D), jnp.float32),
        facc_v=pltpu.VMEM((SEG_PT, D), jnp.float32),
        acc_sh=pltpu.VMEM_SHARED((G, D), jnp.float32),
    ),
)
def _seg_sum(feat_hbm, ids_hbm, sums_hbm, *, ids_v, rows_v, sems, zseg_v,
             facc_v, acc_sh):
    c = lax.axis_index("c")
    s = lax.axis_index("s")
    w = c * NS + s

    zeros16 = jnp.zeros((L,), jnp.float32)
    for i in range(SEG_PT):
        for j in range(D // L):
            zseg_v[i, pl.ds(j * L, L)] = zeros16

    pltpu.sync_copy(zseg_v, acc_sh.at[pl.ds(s * SEG_PT, SEG_PT)])
    plsc.subcore_barrier()

    pltpu.sync_copy(ids_hbm.at[w], ids_v)

    row0 = w * ROWS_PW

    def _perm(i):
        return lax.rem(i * 37, CPT)

    def _feat_chunk(p):
        return feat_hbm.at[pl.ds(row0 + p * CH, CH)]

    def _step(i, cur, cur_sem, nxt, nxt_sem):
        pltpu.make_async_copy(_feat_chunk(_perm(i)), cur, cur_sem).wait()

        @pl.when(i + 1 < CPT)
        def _():
            pltpu.async_copy(_feat_chunk(_perm(i + 1)), nxt, nxt_sem)

        pltpu.sync_copy(cur, acc_sh.at[ids_v.at[_perm(i)]], add=True)

    pltpu.async_copy(_feat_chunk(0), rows_v.at[0], sems.at[0])

    def chunk(i, carry):
        @pl.when(i % 2 == 0)
        def _():
            _step(i, rows_v.at[0], sems.at[0], rows_v.at[1], sems.at[1])

        @pl.when(i % 2 == 1)
        def _():
            _step(i, rows_v.at[1], sems.at[1], rows_v.at[0], sems.at[0])

        return carry

    lax.fori_loop(0, CPT, chunk, 0)
    plsc.subcore_barrier()

    g0 = s * SEG_PT
    pltpu.sync_copy(acc_sh.at[pl.ds(g0, SEG_PT)], facc_v)
    pltpu.sync_copy(facc_v, sums_hbm.at[c, pl.ds(g0, SEG_PT)])


def _hist_body(ids_ref, acc_ref):
    gcol = lax.broadcasted_iota(jnp.int32, (G, 1), 0)

    def blk(r, acc):
        rows = ids_ref[pl.ds(r * HB, HB), :]
        for q in range(HB):
            acc = acc + (gcol == rows[q:q + 1, :]).astype(jnp.float32)
        return acc

    acc_ref[...] = lax.fori_loop(0, IDR // HB, blk,
                                 jnp.zeros((G, D), jnp.float32))


_hist = pl.pallas_call(
    _hist_body,
    out_shape=jax.ShapeDtypeStruct((G, D), jnp.float32),
)


def _finalize_body(sums_ref, acc_ref, out_ref):
    total = sums_ref[0] + sums_ref[1]
    cnt = jnp.sum(acc_ref[...], axis=1, keepdims=True)
    out_ref[...] = total / jnp.maximum(cnt, 1.0)


_finalize = pl.pallas_call(
    _finalize_body,
    out_shape=jax.ShapeDtypeStruct((G, D), jnp.float32),
)


def kernel(feat, segment_ids, num_graphs):
    del num_graphs
    ids = segment_ids.astype(jnp.int32)
    sums = _seg_sum(feat, ids.reshape(NW, CPT, CH))
    acc = _hist(ids.reshape(IDR, D))
    return _finalize(sums, acc)
